# baseline (speedup 1.0000x reference)
import functools

import jax
import jax.numpy as jnp
from jax import lax
from jax.experimental import pallas as pl
from jax.experimental.pallas import tpu as pltpu
from jax.experimental.pallas import tpu_sc as plsc

N = 10000
D = 1024
E = 320000
K = 4
COUT = 128

NW = 32
RPW = 320
NPAD = NW * RPW
NCH = 8
CH = 128
EB = 128


def _make_spmm(has_prev: bool):
    mesh = plsc.VectorSubcoreMesh(core_axis_name="c", subcore_axis_name="s")

    def body(*refs):
        if has_prev:
            (xv_hbm, xprev_hbm, cols_hbm, rows_hbm, vals_hbm, off_hbm,
             y_hbm, a_v, p_v, g_v, col_v, idx_v, row_v, val_v, off_v,
             sem) = refs
        else:
            (xv_hbm, cols_hbm, rows_hbm, vals_hbm, off_hbm,
             y_hbm, a_v, p_v, g_v, col_v, idx_v, row_v, val_v, off_v,
             sem) = refs
            xprev_hbm = None

        w = lax.axis_index("s") * 2 + lax.axis_index("c")
        base = w * RPW
        pltpu.sync_copy(off_hbm, off_v)
        ovec = off_v[pl.ds(w, 16)]
        e0 = ovec[0]
        e1 = ovec[1]
        estart = (e0 // 8) * 8
        nb = (e1 - estart + (EB - 1)) // EB

        def chunk(ch, carry0):
            choff = pl.multiple_of(ch * CH, CH)

            def zero_row(r, carry):
                for j in range(8):
                    a_v[r, pl.ds(j * 16, 16)] = jnp.zeros((16,), jnp.float32)
                return carry
            lax.fori_loop(0, RPW, zero_row, 0)

            def batch(bi, carry):
                eb = estart + bi * EB
                pltpu.sync_copy(cols_hbm.at[pl.ds(eb, EB)], col_v)
                pltpu.sync_copy(rows_hbm.at[pl.ds(eb, EB)], row_v)
                pltpu.sync_copy(vals_hbm.at[pl.ds(eb, EB)], val_v)

                def mkidx(i, c2):
                    idx_v[pl.ds(i * 16, 16)] = col_v[pl.ds(i * 16, 16)] * 8 + ch
                    return c2
                lax.fori_loop(0, EB // 16, mkidx, 0)

                pltpu.async_copy(xv_hbm.at[idx_v], g_v, sem).wait()

                def edge_group(g, c2):
                    rvec = row_v[pl.ds(g * 16, 16)]
                    vvec = val_v[pl.ds(g * 16, 16)]
                    mvec = jnp.logical_and(rvec >= base, rvec < base + RPW)
                    rlvec = jnp.where(mvec, rvec - base, 0)
                    vmvec = jnp.where(mvec, vvec, jnp.float32(0.0))
                    for t in range(16):
                        rl = rlvec[t]
                        vm = vmvec[t]
                        e = g * 16 + t
                        for j in range(8):
                            s = pl.ds(j * 16, 16)
                            plsc.addupdate(a_v.at[rl, s], g_v[e, s] * vm)
                    return c2
                lax.fori_loop(0, EB // 16, edge_group, 0)
                return carry
            lax.fori_loop(0, nb, batch, 0)

            if has_prev:
                pltpu.sync_copy(
                    xprev_hbm.at[pl.ds(base, RPW), pl.ds(choff, CH)], p_v)

                def recur(r, carry):
                    for j in range(8):
                        s = pl.ds(j * 16, 16)
                        a_v[r, s] = 2.0 * a_v[r, s] - p_v[r, s]
                    return carry
                lax.fori_loop(0, RPW, recur, 0)

            pltpu.sync_copy(
                a_v, y_hbm.at[pl.ds(base, RPW), pl.ds(choff, CH)])
            return carry0

        lax.fori_loop(0, NCH, chunk, 0)

    return functools.partial(
        pl.kernel,
        out_type=jax.ShapeDtypeStruct((NPAD, D), jnp.float32),
        mesh=mesh,
        scratch_types=[
            pltpu.VMEM((RPW, CH), jnp.float32),
            pltpu.VMEM((RPW, CH), jnp.float32),
            pltpu.VMEM((EB, CH), jnp.float32),
            pltpu.VMEM((EB,), jnp.int32),
            pltpu.VMEM((EB,), jnp.int32),
            pltpu.VMEM((EB,), jnp.int32),
            pltpu.VMEM((EB,), jnp.float32),
            pltpu.VMEM((48,), jnp.int32),
            pltpu.SemaphoreType.DMA,
        ],
    )(body)


_spmm_first = _make_spmm(has_prev=False)
_spmm_next = _make_spmm(has_prev=True)


def _linear(xts, wts, b2):
    BM = 2000
    nblk = (N * 8) // BM

    def body(x0_ref, x1_ref, x2_ref, x3_ref, w_ref, b_ref, o_ref):
        acc = b_ref[...].astype(jnp.float32) + jnp.zeros((BM, COUT), jnp.float32)
        for t, xr in enumerate((x0_ref, x1_ref, x2_ref, x3_ref)):
            acc = acc + lax.dot_general(
                xr[...], w_ref[t],
                (((1,), (1,)), ((), ())),
                preferred_element_type=jnp.float32)
        o_ref[...] = acc

    return pl.pallas_call(
        body,
        grid=(nblk,),
        in_specs=[pl.BlockSpec((BM, CH), lambda i: (i, 0))] * 4 + [
            pl.BlockSpec((K, COUT, CH), lambda i: (0, 0, 0)),
            pl.BlockSpec((1, COUT), lambda i: (0, 0)),
        ],
        out_specs=pl.BlockSpec((BM, COUT), lambda i: (i, 0)),
        out_shape=jax.ShapeDtypeStruct((N * 8, COUT), jnp.float32),
    )(*xts, wts, b2)


def kernel(x, L_values, L_indices, W, b):
    x0 = x.reshape(N, D)
    x0p = jnp.pad(x0, ((0, NPAD - N), (0, 0)))

    rows = L_indices[0].astype(jnp.int32)
    cols = L_indices[1].astype(jnp.int32)
    rows_s, cols_s, vals_s = lax.sort(
        (rows, cols, L_values), dimension=0, num_keys=1)
    off = jnp.searchsorted(
        rows_s, jnp.arange(NW + 1, dtype=jnp.int32) * RPW).astype(jnp.int32)
    off = jnp.pad(off, (0, 48 - (NW + 1)), constant_values=E)
    pad = EB + 8
    rows_p = jnp.pad(rows_s, (0, pad), constant_values=NPAD)
    cols_p = jnp.pad(cols_s, (0, pad))
    vals_p = jnp.pad(vals_s, (0, pad))

    def gv(a):
        return a.reshape(NPAD * NCH, CH)

    x1 = _spmm_first(gv(x0p), cols_p, rows_p, vals_p, off)
    x2 = _spmm_next(gv(x1), x0p, cols_p, rows_p, vals_p, off)
    x3 = _spmm_next(gv(x2), x1, cols_p, rows_p, vals_p, off)

    wts = W.reshape(COUT, CH, K).transpose(2, 0, 1)
    out = _linear(
        [x0p.reshape(NPAD * 8, CH), x1.reshape(NPAD * 8, CH),
         x2.reshape(NPAD * 8, CH), x3.reshape(NPAD * 8, CH)],
        wts, b.reshape(1, COUT))
    return out.reshape(N, 8, COUT)


# speedup vs baseline: 1.1947x; 1.1947x over previous
import functools

import jax
import jax.numpy as jnp
from jax import lax
from jax.experimental import pallas as pl
from jax.experimental.pallas import tpu as pltpu
from jax.experimental.pallas import tpu_sc as plsc

N = 10000
E = 320000
K = 4
COUT = 128

NW = 32
RPW = 320
NPAD = NW * RPW
NCH = 8
CH = 128
EB = 128
NMB = E // EB + 1


def _make_spmm(has_prev: bool):
    mesh = plsc.VectorSubcoreMesh(core_axis_name="c", subcore_axis_name="s")

    def body(*refs):
        if has_prev:
            (xf_hbm, xprev_hbm, meta_hbm, vals_hbm, off_hbm, y_hbm,
             a_v, p_v, g0_v, g1_v, m0_v, m1_v, v0_v, v1_v, i0_v, i1_v, off_v,
             gsem0, gsem1, msem0, msem1) = refs
        else:
            (xf_hbm, meta_hbm, vals_hbm, off_hbm, y_hbm,
             a_v, p_v, g0_v, g1_v, m0_v, m1_v, v0_v, v1_v, i0_v, i1_v, off_v,
             gsem0, gsem1, msem0, msem1) = refs
            xprev_hbm = None

        w = lax.axis_index("s") * 2 + lax.axis_index("c")
        base = w * RPW
        pltpu.sync_copy(off_hbm, off_v)
        ovec = off_v[pl.ds(w, 16)]
        e0 = ovec[0]
        e1 = ovec[1]
        kb0 = e0 // EB
        nbt = (e1 - kb0 * EB + (EB - 1)) // EB

        def chunk(ch, carry0):
            def zero_row(r, carry):
                for j in range(8):
                    a_v[r, pl.ds(j * 16, 16)] = jnp.zeros((16,), jnp.float32)
                return carry
            lax.fori_loop(0, RPW, zero_row, 0)

            def meta_issue(j, mv, vv, msem):
                pltpu.async_copy(meta_hbm.at[kb0 + j], mv, msem)
                pltpu.async_copy(
                    vals_hbm.at[pl.ds((kb0 + j) * EB, EB)], vv, msem)

            def meta_wait(j, mv, vv, msem):
                pltpu.make_async_copy(meta_hbm.at[kb0 + j], mv, msem).wait()
                pltpu.make_async_copy(
                    vals_hbm.at[pl.ds((kb0 + j) * EB, EB)], vv, msem).wait()

            def gather_fire(mv, iv, gv, gsem):
                def mkidx(i, c2):
                    s = pl.ds(i * 16, 16)
                    iv[s] = mv[0, s] + ch * NPAD
                    return c2
                lax.fori_loop(0, EB // 16, mkidx, 0)
                pltpu.async_copy(xf_hbm.at[iv], gv, gsem)

            def gather_wait(iv, gv, gsem):
                pltpu.make_async_copy(xf_hbm.at[iv], gv, gsem).wait()

            def process(gv, mv, vv):
                def grp(g, c2):
                    s16 = pl.ds(g * 16, 16)
                    rvec = mv[1, s16]
                    vvec = vv[s16]
                    msk = jnp.logical_and(rvec >= base, rvec < base + RPW)
                    rlv = jnp.where(msk, rvec - base, 0)
                    vmv = jnp.where(msk, vvec, jnp.float32(0.0))
                    for t in range(16):
                        rl = rlv[t]
                        vm = vmv[t]
                        for j in range(8):
                            s = pl.ds(j * 16, 16)
                            plsc.addupdate(a_v.at[rl, s], gv[g * 16 + t, s] * vm)
                    return c2
                lax.fori_loop(0, EB // 16, grp, 0)

            bufs = ((g0_v, m0_v, v0_v, i0_v, gsem0, msem0),
                    (g1_v, m1_v, v1_v, i1_v, gsem1, msem1))

            @pl.when(nbt > 0)
            def _():
                meta_issue(0, m0_v, v0_v, msem0)

            @pl.when(nbt > 1)
            def _():
                meta_issue(1, m1_v, v1_v, msem1)

            @pl.when(nbt > 0)
            def _():
                meta_wait(0, m0_v, v0_v, msem0)
                gather_fire(m0_v, i0_v, g0_v, gsem0)

            def pair(p, carry):
                for b in range(2):
                    i = p * 2 + b
                    gv, mv, vv, iv, gsem, msem = bufs[b]
                    go, mo, vo, io, gsemo, msemo = bufs[1 - b]

                    @pl.when(i < nbt)
                    def _():
                        gather_wait(iv, gv, gsem)
                        process(gv, mv, vv)

                        @pl.when(i + 2 < nbt)
                        def _():
                            meta_issue(i + 2, mv, vv, msem)

                        @pl.when(i + 1 < nbt)
                        def _():
                            meta_wait(i + 1, mo, vo, msemo)
                            gather_fire(mo, io, go, gsemo)
                return carry
            lax.fori_loop(0, (nbt + 1) // 2, pair, 0)

            if has_prev:
                pltpu.sync_copy(
                    xprev_hbm.at[ch, pl.ds(base, RPW)], p_v)

                def recur(r, carry):
                    for j in range(8):
                        s = pl.ds(j * 16, 16)
                        a_v[r, s] = 2.0 * a_v[r, s] - p_v[r, s]
                    return carry
                lax.fori_loop(0, RPW, recur, 0)

            pltpu.sync_copy(a_v, y_hbm.at[ch, pl.ds(base, RPW)])
            return carry0

        lax.fori_loop(0, NCH, chunk, 0)

    return functools.partial(
        pl.kernel,
        out_type=jax.ShapeDtypeStruct((NCH, NPAD, CH), jnp.float32),
        mesh=mesh,
        scratch_types=[
            pltpu.VMEM((RPW, CH), jnp.float32),
            pltpu.VMEM((RPW, CH), jnp.float32),
            pltpu.VMEM((EB, CH), jnp.float32),
            pltpu.VMEM((EB, CH), jnp.float32),
            pltpu.VMEM((8, EB), jnp.int32),
            pltpu.VMEM((8, EB), jnp.int32),
            pltpu.VMEM((EB,), jnp.float32),
            pltpu.VMEM((EB,), jnp.float32),
            pltpu.VMEM((EB,), jnp.int32),
            pltpu.VMEM((EB,), jnp.int32),
            pltpu.VMEM((48,), jnp.int32),
            pltpu.SemaphoreType.DMA,
            pltpu.SemaphoreType.DMA,
            pltpu.SemaphoreType.DMA,
            pltpu.SemaphoreType.DMA,
        ],
    )(body)


_spmm_first = _make_spmm(has_prev=False)
_spmm_next = _make_spmm(has_prev=True)


def _linear(xts, wts, b2):
    BN = 400
    nblk = N // BN

    def body(x0_ref, x1_ref, x2_ref, x3_ref, w_ref, b_ref, o_ref):
        acc = b_ref[...].astype(jnp.float32) + jnp.zeros((BN, COUT), jnp.float32)
        for t, xr in enumerate((x0_ref, x1_ref, x2_ref, x3_ref)):
            acc = acc + lax.dot_general(
                xr[0], w_ref[t],
                (((1,), (1,)), ((), ())),
                preferred_element_type=jnp.float32)
        o_ref[...] = acc.reshape(1, BN, COUT)

    return pl.pallas_call(
        body,
        grid=(NCH, nblk),
        in_specs=[pl.BlockSpec((1, BN, CH), lambda f, i: (f, i, 0))] * 4 + [
            pl.BlockSpec((K, COUT, CH), lambda f, i: (0, 0, 0)),
            pl.BlockSpec((1, COUT), lambda f, i: (0, 0)),
        ],
        out_specs=pl.BlockSpec((1, BN, COUT), lambda f, i: (f, i, 0)),
        out_shape=jax.ShapeDtypeStruct((NCH, N, COUT), jnp.float32),
    )(*xts, wts, b2)


def kernel(x, L_values, L_indices, W, b):
    xcm0 = jnp.pad(x.transpose(1, 0, 2), ((0, 0), (0, NPAD - N), (0, 0)))

    rows = L_indices[0].astype(jnp.int32)
    cols = L_indices[1].astype(jnp.int32)
    rows_s, cols_s, vals_s = lax.sort(
        (rows, cols, L_values), dimension=0, num_keys=1)
    off = jnp.searchsorted(
        rows_s, jnp.arange(NW + 1, dtype=jnp.int32) * RPW).astype(jnp.int32)
    off = jnp.pad(off, (0, 48 - (NW + 1)), constant_values=E)

    colsp = jnp.pad(cols_s, (0, EB)).reshape(NMB, EB)
    rowsp = jnp.pad(rows_s, (0, EB), constant_values=NPAD).reshape(NMB, EB)
    valsp = jnp.pad(vals_s, (0, EB))
    meta = (jnp.zeros((NMB, 8, EB), jnp.int32)
            .at[:, 0, :].set(colsp)
            .at[:, 1, :].set(rowsp))

    def fv(a):
        return a.reshape(NCH * NPAD, CH)

    x1 = _spmm_first(fv(xcm0), meta, valsp, off)
    x2 = _spmm_next(fv(x1), xcm0, meta, valsp, off)
    x3 = _spmm_next(fv(x2), x1, meta, valsp, off)

    wts = W.reshape(COUT, CH, K).transpose(2, 0, 1)
    out = _linear([xcm0, x1, x2, x3], wts, b.reshape(1, COUT))
    return out.transpose(1, 0, 2)


# speedup vs baseline: 2.5394x; 2.1256x over previous
import functools

import jax
import jax.numpy as jnp
from jax import lax
from jax.experimental import pallas as pl
from jax.experimental.pallas import tpu as pltpu
from jax.experimental.pallas import tpu_sc as plsc

N = 10000
E = 320000
K = 4
COUT = 128

NW = 32
RPW = 320
NPAD = NW * RPW
NCH = 8
CH = 128
EB = 128
NMB = E // EB + 1


def _make_spmm(has_prev: bool):
    mesh = plsc.VectorSubcoreMesh(core_axis_name="c", subcore_axis_name="s")

    def body(*refs):
        if has_prev:
            (xf_hbm, xprev_hbm, meta_hbm, vals_hbm, off_hbm, y_hbm,
             a_v, p_v, g0_v, g1_v, m0_v, m1_v, v0_v, v1_v, i0_v, i1_v, off_v,
             gsem0, gsem1, msem0, msem1) = refs
        else:
            (xf_hbm, meta_hbm, vals_hbm, off_hbm, y_hbm,
             a_v, p_v, g0_v, g1_v, m0_v, m1_v, v0_v, v1_v, i0_v, i1_v, off_v,
             gsem0, gsem1, msem0, msem1) = refs
            xprev_hbm = None

        w = lax.axis_index("s") * 2 + lax.axis_index("c")
        base = w * RPW
        pltpu.sync_copy(off_hbm, off_v)
        ovec = off_v[pl.ds(w, 16)]
        e0 = ovec[0]
        e1 = ovec[1]
        kb0 = e0 // EB
        nbt = (e1 - kb0 * EB + (EB - 1)) // EB

        def chunk(ch, carry0):
            def zero_row(r, carry):
                for j in range(8):
                    a_v[r, pl.ds(j * 16, 16)] = jnp.zeros((16,), jnp.float32)
                return carry
            lax.fori_loop(0, RPW, zero_row, 0)

            def meta_issue(j, mv, vv, msem):
                pltpu.async_copy(meta_hbm.at[kb0 + j], mv, msem)
                pltpu.async_copy(
                    vals_hbm.at[pl.ds((kb0 + j) * EB, EB)], vv, msem)

            def meta_wait(j, mv, vv, msem):
                pltpu.make_async_copy(meta_hbm.at[kb0 + j], mv, msem).wait()
                pltpu.make_async_copy(
                    vals_hbm.at[pl.ds((kb0 + j) * EB, EB)], vv, msem).wait()

            def gather_fire(mv, iv, gv, gsem):
                def mkidx(i, c2):
                    s = pl.ds(i * 16, 16)
                    iv[s] = mv[0, s] + ch * NPAD
                    return c2
                lax.fori_loop(0, EB // 16, mkidx, 0)
                pltpu.async_copy(xf_hbm.at[iv], gv, gsem)

            def gather_wait(iv, gv, gsem):
                pltpu.make_async_copy(xf_hbm.at[iv], gv, gsem).wait()

            def process(gv, mv, vv):
                def grp(g, c2):
                    s16 = pl.ds(g * 16, 16)
                    rvec = mv[1, s16]
                    vvec = vv[s16]
                    msk = jnp.logical_and(rvec >= base, rvec < base + RPW)
                    rlv = jnp.where(msk, rvec - base, 0)
                    vmv = jnp.where(msk, vvec, jnp.float32(0.0))
                    for t in range(16):
                        rl = rlv[t]
                        vm = vmv[t]
                        e = g * 16 + t
                        gs = [gv[e, pl.ds(j * 16, 16)] for j in range(8)]
                        for j in range(8):
                            plsc.addupdate(
                                a_v.at[rl, pl.ds(j * 16, 16)], gs[j] * vm)
                    return c2
                lax.fori_loop(0, EB // 16, grp, 0)

            bufs = ((g0_v, m0_v, v0_v, i0_v, gsem0, msem0),
                    (g1_v, m1_v, v1_v, i1_v, gsem1, msem1))

            @pl.when(nbt > 0)
            def _():
                meta_issue(0, m0_v, v0_v, msem0)

            @pl.when(nbt > 1)
            def _():
                meta_issue(1, m1_v, v1_v, msem1)

            @pl.when(nbt > 0)
            def _():
                meta_wait(0, m0_v, v0_v, msem0)
                gather_fire(m0_v, i0_v, g0_v, gsem0)

            def pair(p, carry):
                for b in range(2):
                    i = p * 2 + b
                    gv, mv, vv, iv, gsem, msem = bufs[b]
                    go, mo, vo, io, gsemo, msemo = bufs[1 - b]

                    @pl.when(i < nbt)
                    def _():
                        gather_wait(iv, gv, gsem)
                        process(gv, mv, vv)

                        @pl.when(i + 2 < nbt)
                        def _():
                            meta_issue(i + 2, mv, vv, msem)

                        @pl.when(i + 1 < nbt)
                        def _():
                            meta_wait(i + 1, mo, vo, msemo)
                            gather_fire(mo, io, go, gsemo)
                return carry
            lax.fori_loop(0, (nbt + 1) // 2, pair, 0)

            if has_prev:
                pltpu.sync_copy(
                    xprev_hbm.at[ch, pl.ds(base, RPW)], p_v)

                def recur(r, carry):
                    avs = [a_v[r, pl.ds(j * 16, 16)] for j in range(8)]
                    pvs = [p_v[r, pl.ds(j * 16, 16)] for j in range(8)]
                    for j in range(8):
                        a_v[r, pl.ds(j * 16, 16)] = 2.0 * avs[j] - pvs[j]
                    return carry
                lax.fori_loop(0, RPW, recur, 0)

            pltpu.sync_copy(a_v, y_hbm.at[ch, pl.ds(base, RPW)])
            return carry0

        lax.fori_loop(0, NCH, chunk, 0)

    return functools.partial(
        pl.kernel,
        out_type=jax.ShapeDtypeStruct((NCH, NPAD, CH), jnp.float32),
        mesh=mesh,
        scratch_types=[
            pltpu.VMEM((RPW, CH), jnp.float32),
            pltpu.VMEM((RPW, CH), jnp.float32),
            pltpu.VMEM((EB, CH), jnp.float32),
            pltpu.VMEM((EB, CH), jnp.float32),
            pltpu.VMEM((8, EB), jnp.int32),
            pltpu.VMEM((8, EB), jnp.int32),
            pltpu.VMEM((EB,), jnp.float32),
            pltpu.VMEM((EB,), jnp.float32),
            pltpu.VMEM((EB,), jnp.int32),
            pltpu.VMEM((EB,), jnp.int32),
            pltpu.VMEM((48,), jnp.int32),
            pltpu.SemaphoreType.DMA,
            pltpu.SemaphoreType.DMA,
            pltpu.SemaphoreType.DMA,
            pltpu.SemaphoreType.DMA,
        ],
    )(body)


_spmm_first = _make_spmm(has_prev=False)
_spmm_next = _make_spmm(has_prev=True)


def _linear(xts, wts, b2):
    BN = 400
    nblk = N // BN

    def body(x0_ref, x1_ref, x2_ref, x3_ref, w_ref, b_ref, o_ref):
        acc = b_ref[...].astype(jnp.float32) + jnp.zeros((BN, COUT), jnp.float32)
        for t, xr in enumerate((x0_ref, x1_ref, x2_ref, x3_ref)):
            acc = acc + lax.dot_general(
                xr[0], w_ref[t],
                (((1,), (1,)), ((), ())),
                preferred_element_type=jnp.float32)
        o_ref[...] = acc.reshape(1, BN, COUT)

    return pl.pallas_call(
        body,
        grid=(NCH, nblk),
        in_specs=[pl.BlockSpec((1, BN, CH), lambda f, i: (f, i, 0))] * 4 + [
            pl.BlockSpec((K, COUT, CH), lambda f, i: (0, 0, 0)),
            pl.BlockSpec((1, COUT), lambda f, i: (0, 0)),
        ],
        out_specs=pl.BlockSpec((1, BN, COUT), lambda f, i: (f, i, 0)),
        out_shape=jax.ShapeDtypeStruct((NCH, N, COUT), jnp.float32),
    )(*xts, wts, b2)


def kernel(x, L_values, L_indices, W, b):
    xcm0 = jnp.pad(x.transpose(1, 0, 2), ((0, 0), (0, NPAD - N), (0, 0)))

    rows = L_indices[0].astype(jnp.int32)
    cols = L_indices[1].astype(jnp.int32)
    rows_s, cols_s, vals_s = lax.sort(
        (rows, cols, L_values), dimension=0, num_keys=1)
    off = jnp.searchsorted(
        rows_s, jnp.arange(NW + 1, dtype=jnp.int32) * RPW).astype(jnp.int32)
    off = jnp.pad(off, (0, 48 - (NW + 1)), constant_values=E)

    colsp = jnp.pad(cols_s, (0, EB)).reshape(NMB, EB)
    rowsp = jnp.pad(rows_s, (0, EB), constant_values=NPAD).reshape(NMB, EB)
    valsp = jnp.pad(vals_s, (0, EB))
    meta = (jnp.zeros((NMB, 8, EB), jnp.int32)
            .at[:, 0, :].set(colsp)
            .at[:, 1, :].set(rowsp))

    def fv(a):
        return a.reshape(NCH * NPAD, CH)

    x1 = _spmm_first(fv(xcm0), meta, valsp, off)
    x2 = _spmm_next(fv(x1), xcm0, meta, valsp, off)
    x3 = _spmm_next(fv(x2), x1, meta, valsp, off)

    wts = W.reshape(COUT, CH, K).transpose(2, 0, 1)
    out = _linear([xcm0, x1, x2, x3], wts, b.reshape(1, COUT))
    return out.transpose(1, 0, 2)


# speedup vs baseline: 4.0938x; 1.6121x over previous
import functools

import jax
import jax.numpy as jnp
from jax import lax
from jax.experimental import pallas as pl
from jax.experimental.pallas import tpu as pltpu
from jax.experimental.pallas import tpu_sc as plsc

N = 10000
E = 320000
K = 4
COUT = 128

NW = 32
RPW = 320
NPAD = NW * RPW
NCH = 8
CH = 128
EB = 128
NMB = E // EB + 1


def _make_spmm(has_prev: bool):
    mesh = plsc.VectorSubcoreMesh(core_axis_name="c", subcore_axis_name="s")

    def body(*refs):
        if has_prev:
            (xf_hbm, xprev_hbm, meta_hbm, vals_hbm, off_hbm, y_hbm,
             a_v, p_v, g0_v, g1_v, g2_v, m0_v, m1_v, m2_v,
             v0_v, v1_v, v2_v, i0_v, i1_v, i2_v, off_v,
             gsem0, gsem1, gsem2, msem0, msem1, msem2) = refs
        else:
            (xf_hbm, meta_hbm, vals_hbm, off_hbm, y_hbm,
             a_v, p_v, g0_v, g1_v, g2_v, m0_v, m1_v, m2_v,
             v0_v, v1_v, v2_v, i0_v, i1_v, i2_v, off_v,
             gsem0, gsem1, gsem2, msem0, msem1, msem2) = refs
            xprev_hbm = None

        w = lax.axis_index("s") * 2 + lax.axis_index("c")
        base = w * RPW
        pltpu.sync_copy(off_hbm, off_v)
        ovec = off_v[pl.ds(w, 16)]
        e0 = ovec[0]
        e1 = ovec[1]
        kb0 = e0 // EB
        nbt = (e1 - kb0 * EB + (EB - 1)) // EB

        def chunk(ch, carry0):
            def zero_row(r, carry):
                for j in range(8):
                    a_v[r, pl.ds(j * 16, 16)] = jnp.zeros((16,), jnp.float32)
                return carry
            lax.fori_loop(0, RPW, zero_row, 0)

            def meta_issue(j, mv, vv, msem):
                pltpu.async_copy(meta_hbm.at[kb0 + j], mv, msem)
                pltpu.async_copy(
                    vals_hbm.at[pl.ds((kb0 + j) * EB, EB)], vv, msem)

            def meta_wait(j, mv, vv, msem):
                pltpu.make_async_copy(meta_hbm.at[kb0 + j], mv, msem).wait()
                pltpu.make_async_copy(
                    vals_hbm.at[pl.ds((kb0 + j) * EB, EB)], vv, msem).wait()

            def gather_fire(mv, iv, gv, gsem):
                def mkidx(i, c2):
                    s = pl.ds(i * 16, 16)
                    iv[s] = mv[0, s] + ch * NPAD
                    return c2
                lax.fori_loop(0, EB // 16, mkidx, 0)
                pltpu.async_copy(xf_hbm.at[iv], gv, gsem)

            def gather_wait(iv, gv, gsem):
                pltpu.make_async_copy(xf_hbm.at[iv], gv, gsem).wait()

            def process(gv, mv, vv):
                def grp(g, c2):
                    s16 = pl.ds(g * 16, 16)
                    rvec = mv[1, s16]
                    vvec = vv[s16]
                    msk = jnp.logical_and(rvec >= base, rvec < base + RPW)
                    rlv = jnp.where(msk, rvec - base, 0)
                    vmv = jnp.where(msk, vvec, jnp.float32(0.0))
                    for tb in range(8):
                        ts = (tb * 2, tb * 2 + 1)
                        rls = [rlv[t] for t in ts]
                        vms = [vmv[t] for t in ts]
                        gss = [[gv[g * 16 + t, pl.ds(j * 16, 16)]
                                for j in range(8)] for t in ts]
                        for u in range(2):
                            for j in range(8):
                                plsc.addupdate(
                                    a_v.at[rls[u], pl.ds(j * 16, 16)],
                                    gss[u][j] * vms[u])
                    return c2
                lax.fori_loop(0, EB // 16, grp, 0)

            bufs = ((g0_v, m0_v, v0_v, i0_v, gsem0, msem0),
                    (g1_v, m1_v, v1_v, i1_v, gsem1, msem1),
                    (g2_v, m2_v, v2_v, i2_v, gsem2, msem2))

            @pl.when(nbt > 0)
            def _():
                meta_issue(0, m0_v, v0_v, msem0)

            @pl.when(nbt > 1)
            def _():
                meta_issue(1, m1_v, v1_v, msem1)

            @pl.when(nbt > 0)
            def _():
                meta_wait(0, m0_v, v0_v, msem0)
                gather_fire(m0_v, i0_v, g0_v, gsem0)

            def triple(p, carry):
                for b in range(3):
                    i = p * 3 + b
                    gv, mv, vv, iv, gsem, msem = bufs[b]
                    gn, mn, vn, inn, gsemn, msemn = bufs[(b + 1) % 3]
                    g2, m2, v2, i2, gsem2_, msem2_ = bufs[(b + 2) % 3]

                    @pl.when(i < nbt)
                    def _():
                        gather_wait(iv, gv, gsem)

                        @pl.when(i + 1 < nbt)
                        def _():
                            meta_wait(i + 1, mn, vn, msemn)
                            gather_fire(mn, inn, gn, gsemn)

                        @pl.when(i + 2 < nbt)
                        def _():
                            meta_issue(i + 2, m2, v2, msem2_)

                        process(gv, mv, vv)
                return carry
            lax.fori_loop(0, (nbt + 2) // 3, triple, 0)

            if has_prev:
                for q in range(4):
                    pltpu.sync_copy(
                        xprev_hbm.at[ch, pl.ds(base + q * (RPW // 4),
                                               RPW // 4)], p_v)

                    def recur(r, carry):
                        ar = q * (RPW // 4) + r
                        avs = [a_v[ar, pl.ds(j * 16, 16)] for j in range(8)]
                        pvs = [p_v[r, pl.ds(j * 16, 16)] for j in range(8)]
                        for j in range(8):
                            a_v[ar, pl.ds(j * 16, 16)] = 2.0 * avs[j] - pvs[j]
                        return carry
                    lax.fori_loop(0, RPW // 4, recur, 0)

            pltpu.sync_copy(a_v, y_hbm.at[ch, pl.ds(base, RPW)])
            return carry0

        lax.fori_loop(0, NCH, chunk, 0)

    return functools.partial(
        pl.kernel,
        out_type=jax.ShapeDtypeStruct((NCH, NPAD, CH), jnp.float32),
        mesh=mesh,
        scratch_types=[
            pltpu.VMEM((RPW, CH), jnp.float32),
            pltpu.VMEM((RPW // 4, CH), jnp.float32),
            pltpu.VMEM((EB, CH), jnp.float32),
            pltpu.VMEM((EB, CH), jnp.float32),
            pltpu.VMEM((EB, CH), jnp.float32),
            pltpu.VMEM((8, EB), jnp.int32),
            pltpu.VMEM((8, EB), jnp.int32),
            pltpu.VMEM((8, EB), jnp.int32),
            pltpu.VMEM((EB,), jnp.float32),
            pltpu.VMEM((EB,), jnp.float32),
            pltpu.VMEM((EB,), jnp.float32),
            pltpu.VMEM((EB,), jnp.int32),
            pltpu.VMEM((EB,), jnp.int32),
            pltpu.VMEM((EB,), jnp.int32),
            pltpu.VMEM((48,), jnp.int32),
            pltpu.SemaphoreType.DMA,
            pltpu.SemaphoreType.DMA,
            pltpu.SemaphoreType.DMA,
            pltpu.SemaphoreType.DMA,
            pltpu.SemaphoreType.DMA,
            pltpu.SemaphoreType.DMA,
        ],
    )(body)


_spmm_first = _make_spmm(has_prev=False)
_spmm_next = _make_spmm(has_prev=True)


def _linear(xts, wts, b2):
    BN = 400
    nblk = N // BN

    def body(x0_ref, x1_ref, x2_ref, x3_ref, w_ref, b_ref, o_ref):
        acc = b_ref[...].astype(jnp.float32) + jnp.zeros((BN, COUT), jnp.float32)
        for t, xr in enumerate((x0_ref, x1_ref, x2_ref, x3_ref)):
            acc = acc + lax.dot_general(
                xr[0], w_ref[t],
                (((1,), (1,)), ((), ())),
                preferred_element_type=jnp.float32)
        o_ref[...] = acc.reshape(1, BN, COUT)

    return pl.pallas_call(
        body,
        grid=(NCH, nblk),
        in_specs=[pl.BlockSpec((1, BN, CH), lambda f, i: (f, i, 0))] * 4 + [
            pl.BlockSpec((K, COUT, CH), lambda f, i: (0, 0, 0)),
            pl.BlockSpec((1, COUT), lambda f, i: (0, 0)),
        ],
        out_specs=pl.BlockSpec((1, BN, COUT), lambda f, i: (f, i, 0)),
        out_shape=jax.ShapeDtypeStruct((NCH, N, COUT), jnp.float32),
    )(*xts, wts, b2)


def kernel(x, L_values, L_indices, W, b):
    xcm0 = jnp.pad(x.transpose(1, 0, 2), ((0, 0), (0, NPAD - N), (0, 0)))

    rows = L_indices[0].astype(jnp.int32)
    cols = L_indices[1].astype(jnp.int32)
    rows_s, cols_s, vals_s = lax.sort(
        (rows, cols, L_values), dimension=0, num_keys=1)
    off = jnp.searchsorted(
        rows_s, jnp.arange(NW + 1, dtype=jnp.int32) * RPW).astype(jnp.int32)
    off = jnp.pad(off, (0, 48 - (NW + 1)), constant_values=E)

    colsp = jnp.pad(cols_s, (0, EB)).reshape(NMB, EB)
    rowsp = jnp.pad(rows_s, (0, EB), constant_values=NPAD).reshape(NMB, EB)
    valsp = jnp.pad(vals_s, (0, EB))
    meta = (jnp.zeros((NMB, 8, EB), jnp.int32)
            .at[:, 0, :].set(colsp)
            .at[:, 1, :].set(rowsp))

    def fv(a):
        return a.reshape(NCH * NPAD, CH)

    x1 = _spmm_first(fv(xcm0), meta, valsp, off)
    x2 = _spmm_next(fv(x1), xcm0, meta, valsp, off)
    x3 = _spmm_next(fv(x2), x1, meta, valsp, off)

    wts = W.reshape(COUT, CH, K).transpose(2, 0, 1)
    out = _linear([xcm0, x1, x2, x3], wts, b.reshape(1, COUT))
    return out.transpose(1, 0, 2)


# speedup vs baseline: 4.2656x; 1.0420x over previous
import functools

import jax
import jax.numpy as jnp
from jax import lax
from jax.experimental import pallas as pl
from jax.experimental.pallas import tpu as pltpu
from jax.experimental.pallas import tpu_sc as plsc

N = 10000
E = 320000
K = 4
COUT = 128

NW = 32
RPW = 320
NPAD = NW * RPW
NCH = 8
CH = 128
EB = 128
NMB = E // EB + 1
CBITS = 14
CMASK = (1 << CBITS) - 1


def _make_spmm(has_prev: bool):
    mesh = plsc.VectorSubcoreMesh(core_axis_name="c", subcore_axis_name="s")

    def body(*refs):
        if has_prev:
            (xb_hbm, xprev_hbm, pk_hbm, vals_hbm, off_hbm, y_hbm,
             a_v, p_v, g0_v, g1_v, g2_v, k0_v, k1_v, k2_v,
             v0_v, v1_v, v2_v, i0_v, i1_v, i2_v, off_v,
             gsem0, gsem1, gsem2, msem0, msem1, msem2) = refs
        else:
            (xb_hbm, pk_hbm, vals_hbm, off_hbm, y_hbm,
             a_v, p_v, g0_v, g1_v, g2_v, k0_v, k1_v, k2_v,
             v0_v, v1_v, v2_v, i0_v, i1_v, i2_v, off_v,
             gsem0, gsem1, gsem2, msem0, msem1, msem2) = refs
            xprev_hbm = None

        w = lax.axis_index("s") * 2 + lax.axis_index("c")
        base = w * RPW
        pltpu.sync_copy(off_hbm, off_v)
        ovec = off_v[pl.ds(w, 16)]
        e0 = ovec[0]
        e1 = ovec[1]
        kb0 = e0 // EB
        nbt = (e1 - kb0 * EB + (EB - 1)) // EB

        def chunk(ch, carry0):
            def zero_row(r, carry):
                for j in range(8):
                    a_v[r, pl.ds(j * 16, 16)] = jnp.zeros((16,), jnp.float32)
                return carry
            lax.fori_loop(0, RPW, zero_row, 0)

            def meta_issue(j, kv, vv, msem):
                pltpu.async_copy(
                    pk_hbm.at[pl.ds((kb0 + j) * EB, EB)], kv, msem)
                pltpu.async_copy(
                    vals_hbm.at[pl.ds((kb0 + j) * EB, EB)], vv, msem)

            def meta_wait(j, kv, vv, msem):
                pltpu.make_async_copy(
                    pk_hbm.at[pl.ds((kb0 + j) * EB, EB)], kv, msem).wait()
                pltpu.make_async_copy(
                    vals_hbm.at[pl.ds((kb0 + j) * EB, EB)], vv, msem).wait()

            def gather_fire(kv, iv, gv, gsem):
                def mkidx(i, c2):
                    s = pl.ds(i * 16, 16)
                    iv[s] = (kv[s] & CMASK) + ch * NPAD
                    return c2
                lax.fori_loop(0, EB // 16, mkidx, 0)
                pltpu.async_copy(xb_hbm.at[iv], gv, gsem)

            def gather_wait(iv, gv, gsem):
                pltpu.make_async_copy(xb_hbm.at[iv], gv, gsem).wait()

            def process(gv, kv, vv):
                def grp(g, c2):
                    s16 = pl.ds(g * 16, 16)
                    ebase = pl.multiple_of(g * 16, 16)
                    rvec = lax.shift_right_logical(kv[s16], CBITS)
                    msk = jnp.logical_and(rvec >= base, rvec < base + RPW)
                    rlv = jnp.where(msk, rvec - base, 0)
                    vmv = jnp.where(msk, vv[s16], jnp.float32(0.0))
                    for tb in range(8):
                        ts = (tb * 2, tb * 2 + 1)
                        rls = [rlv[t] for t in ts]
                        vms = [vmv[t] for t in ts]
                        gss = [[gv[ebase + t, pl.ds(j * 16, 16)]
                                for j in range(8)] for t in ts]
                        for u in range(2):
                            for j in range(8):
                                plsc.addupdate(
                                    a_v.at[rls[u], pl.ds(j * 16, 16)],
                                    gss[u][j] * vms[u])
                    return c2
                lax.fori_loop(0, EB // 16, grp, 0)

            bufs = ((g0_v, k0_v, v0_v, i0_v, gsem0, msem0),
                    (g1_v, k1_v, v1_v, i1_v, gsem1, msem1),
                    (g2_v, k2_v, v2_v, i2_v, gsem2, msem2))

            @pl.when(nbt > 0)
            def _():
                meta_issue(0, k0_v, v0_v, msem0)

            @pl.when(nbt > 1)
            def _():
                meta_issue(1, k1_v, v1_v, msem1)

            @pl.when(nbt > 0)
            def _():
                meta_wait(0, k0_v, v0_v, msem0)
                gather_fire(k0_v, i0_v, g0_v, gsem0)

            def triple(p, carry):
                for b in range(3):
                    i = p * 3 + b
                    gv, kv, vv, iv, gsem, msem = bufs[b]
                    gn, kn, vn, inn, gsemn, msemn = bufs[(b + 1) % 3]
                    gl, kl, vl, il, gseml, mseml = bufs[(b + 2) % 3]

                    @pl.when(i < nbt)
                    def _():
                        gather_wait(iv, gv, gsem)

                        @pl.when(i + 1 < nbt)
                        def _():
                            meta_wait(i + 1, kn, vn, msemn)
                            gather_fire(kn, inn, gn, gsemn)

                        @pl.when(i + 2 < nbt)
                        def _():
                            meta_issue(i + 2, kl, vl, mseml)

                        process(gv, kv, vv)
                return carry
            lax.fori_loop(0, (nbt + 2) // 3, triple, 0)

            if has_prev:
                for q in range(4):
                    pltpu.sync_copy(
                        xprev_hbm.at[ch, pl.ds(base + q * (RPW // 4),
                                               RPW // 4)], p_v)

                    def recur(r, carry):
                        ar = q * (RPW // 4) + r
                        avs = [a_v[ar, pl.ds(j * 16, 16)] for j in range(8)]
                        pvs = [p_v[r, pl.ds(j * 16, 16)] for j in range(8)]
                        for j in range(8):
                            a_v[ar, pl.ds(j * 16, 16)] = 2.0 * avs[j] - pvs[j]
                        return carry
                    lax.fori_loop(0, RPW // 4, recur, 0)

            pltpu.sync_copy(a_v, y_hbm.at[ch, pl.ds(base, RPW)])
            return carry0

        lax.fori_loop(0, NCH, chunk, 0)

    return functools.partial(
        pl.kernel,
        out_type=jax.ShapeDtypeStruct((NCH, NPAD, CH), jnp.float32),
        mesh=mesh,
        scratch_types=[
            pltpu.VMEM((RPW, CH), jnp.float32),
            pltpu.VMEM((RPW // 4, CH), jnp.float32),
            pltpu.VMEM((EB, CH), jnp.float32),
            pltpu.VMEM((EB, CH), jnp.float32),
            pltpu.VMEM((EB, CH), jnp.float32),
            pltpu.VMEM((EB,), jnp.int32),
            pltpu.VMEM((EB,), jnp.int32),
            pltpu.VMEM((EB,), jnp.int32),
            pltpu.VMEM((EB,), jnp.float32),
            pltpu.VMEM((EB,), jnp.float32),
            pltpu.VMEM((EB,), jnp.float32),
            pltpu.VMEM((EB,), jnp.int32),
            pltpu.VMEM((EB,), jnp.int32),
            pltpu.VMEM((EB,), jnp.int32),
            pltpu.VMEM((48,), jnp.int32),
            pltpu.SemaphoreType.DMA,
            pltpu.SemaphoreType.DMA,
            pltpu.SemaphoreType.DMA,
            pltpu.SemaphoreType.DMA,
            pltpu.SemaphoreType.DMA,
            pltpu.SemaphoreType.DMA,
        ],
    )(body)


_spmm_first = _make_spmm(has_prev=False)
_spmm_next = _make_spmm(has_prev=True)


def _linear(xts, wts, b2):
    BN = 400
    nblk = N // BN

    def body(x0_ref, x1_ref, x2_ref, x3_ref, w_ref, b_ref, o_ref):
        acc = b_ref[...].astype(jnp.float32) + jnp.zeros((BN, COUT), jnp.float32)
        for t, xr in enumerate((x0_ref, x1_ref, x2_ref, x3_ref)):
            acc = acc + lax.dot_general(
                xr[0], w_ref[t],
                (((1,), (1,)), ((), ())),
                preferred_element_type=jnp.float32)
        o_ref[...] = acc.reshape(1, BN, COUT)

    return pl.pallas_call(
        body,
        grid=(NCH, nblk),
        in_specs=[pl.BlockSpec((1, BN, CH), lambda f, i: (f, i, 0))] * 4 + [
            pl.BlockSpec((K, COUT, CH), lambda f, i: (0, 0, 0)),
            pl.BlockSpec((1, COUT), lambda f, i: (0, 0)),
        ],
        out_specs=pl.BlockSpec((1, BN, COUT), lambda f, i: (f, i, 0)),
        out_shape=jax.ShapeDtypeStruct((NCH, N, COUT), jnp.float32),
    )(*xts, wts, b2)


def _gather_view(xcm):
    return xcm.reshape(NCH * NPAD, CH)


def kernel(x, L_values, L_indices, W, b):
    xcm0 = jnp.pad(x.transpose(1, 0, 2), ((0, 0), (0, NPAD - N), (0, 0)))

    rows = L_indices[0].astype(jnp.int32)
    cols = L_indices[1].astype(jnp.int32)
    packed = (rows << CBITS) | cols
    pk_s, vals_s = lax.sort((packed, L_values), dimension=0, num_keys=1)
    off = jnp.searchsorted(
        pk_s, jnp.arange(NW + 1, dtype=jnp.int32) * (RPW << CBITS)
    ).astype(jnp.int32)
    off = jnp.pad(off, (0, 48 - (NW + 1)), constant_values=E)

    pk_p = jnp.pad(pk_s, (0, EB), constant_values=NPAD << CBITS)
    vals_p = jnp.pad(vals_s, (0, EB))

    x1 = _spmm_first(_gather_view(xcm0), pk_p, vals_p, off)
    x2 = _spmm_next(_gather_view(x1), xcm0, pk_p, vals_p, off)
    x3 = _spmm_next(_gather_view(x2), x1, pk_p, vals_p, off)

    wts = W.reshape(COUT, CH, K).transpose(2, 0, 1)
    out = _linear([xcm0, x1, x2, x3], wts, b.reshape(1, COUT))
    return out.transpose(1, 0, 2)


# speedup vs baseline: 4.2830x; 1.0041x over previous
import functools

import jax
import jax.numpy as jnp
from jax import lax
from jax.experimental import pallas as pl
from jax.experimental.pallas import tpu as pltpu
from jax.experimental.pallas import tpu_sc as plsc

N = 10000
E = 320000
K = 4
COUT = 128

NW = 32
RPW = 320
NPAD = NW * RPW
NCH = 8
CH = 128
EB = 128
NMB = E // EB + 1
CBITS = 14
CMASK = (1 << CBITS) - 1


def _make_spmm(has_prev: bool):
    mesh = plsc.VectorSubcoreMesh(core_axis_name="c", subcore_axis_name="s")

    def body(*refs):
        if has_prev:
            (xb_hbm, xprev_hbm, pk_hbm, vals_hbm, off_hbm, y_hbm,
             a_v, p_v, g0_v, g1_v, g2_v, k0_v, k1_v, k2_v,
             v0_v, v1_v, v2_v, i0_v, i1_v, i2_v, off_v,
             gsem0, gsem1, gsem2, msem0, msem1, msem2) = refs
        else:
            (xb_hbm, pk_hbm, vals_hbm, off_hbm, y_hbm,
             a_v, p_v, g0_v, g1_v, g2_v, k0_v, k1_v, k2_v,
             v0_v, v1_v, v2_v, i0_v, i1_v, i2_v, off_v,
             gsem0, gsem1, gsem2, msem0, msem1, msem2) = refs
            xprev_hbm = None

        w = lax.axis_index("s") * 2 + lax.axis_index("c")
        base = w * RPW
        pltpu.sync_copy(off_hbm, off_v)
        ovec = off_v[pl.ds(w, 16)]
        e0 = ovec[0]
        e1 = ovec[1]
        kb0 = e0 // EB
        nbt = (e1 - kb0 * EB + (EB - 1)) // EB

        def chunk(ch, carry0):
            def zero_row(r, carry):
                for j in range(8):
                    a_v[r, pl.ds(j * 16, 16)] = jnp.zeros((16,), jnp.float32)
                return carry
            lax.fori_loop(0, RPW, zero_row, 0)

            def meta_issue(j, kv, vv, msem):
                pltpu.async_copy(
                    pk_hbm.at[pl.ds((kb0 + j) * EB, EB)], kv, msem)
                pltpu.async_copy(
                    vals_hbm.at[pl.ds((kb0 + j) * EB, EB)], vv, msem)

            def meta_wait(j, kv, vv, msem):
                pltpu.make_async_copy(
                    pk_hbm.at[pl.ds((kb0 + j) * EB, EB)], kv, msem).wait()
                pltpu.make_async_copy(
                    vals_hbm.at[pl.ds((kb0 + j) * EB, EB)], vv, msem).wait()

            def gather_fire(kv, iv, gv, gsem):
                def mkidx(i, c2):
                    s = pl.ds(i * 16, 16)
                    iv[s] = (kv[s] & CMASK) + ch * NPAD
                    return c2
                lax.fori_loop(0, EB // 16, mkidx, 0)
                pltpu.async_copy(xb_hbm.at[iv], gv, gsem)

            def gather_wait(iv, gv, gsem):
                pltpu.make_async_copy(xb_hbm.at[iv], gv, gsem).wait()

            def process(gv, kv, vv):
                def grp(g, c2):
                    s16 = pl.ds(g * 16, 16)
                    ebase = pl.multiple_of(g * 16, 16)
                    rvec = lax.shift_right_logical(kv[s16], CBITS)
                    msk = jnp.logical_and(rvec >= base, rvec < base + RPW)
                    rlv = jnp.where(msk, rvec - base, 0)
                    vmv = jnp.where(msk, vv[s16], jnp.float32(0.0))
                    for tb in range(4):
                        ts = (tb * 4, tb * 4 + 1, tb * 4 + 2, tb * 4 + 3)
                        rls = [rlv[t] for t in ts]
                        vms = [vmv[t] for t in ts]
                        gss = [[gv[ebase + t, pl.ds(j * 16, 16)]
                                for j in range(8)] for t in ts]
                        for u in range(4):
                            for j in range(8):
                                plsc.addupdate(
                                    a_v.at[rls[u], pl.ds(j * 16, 16)],
                                    gss[u][j] * vms[u])
                    return c2
                lax.fori_loop(0, EB // 16, grp, 0)

            bufs = ((g0_v, k0_v, v0_v, i0_v, gsem0, msem0),
                    (g1_v, k1_v, v1_v, i1_v, gsem1, msem1),
                    (g2_v, k2_v, v2_v, i2_v, gsem2, msem2))

            @pl.when(nbt > 0)
            def _():
                meta_issue(0, k0_v, v0_v, msem0)

            @pl.when(nbt > 1)
            def _():
                meta_issue(1, k1_v, v1_v, msem1)

            @pl.when(nbt > 0)
            def _():
                meta_wait(0, k0_v, v0_v, msem0)
                gather_fire(k0_v, i0_v, g0_v, gsem0)

            def triple(p, carry):
                for b in range(3):
                    i = p * 3 + b
                    gv, kv, vv, iv, gsem, msem = bufs[b]
                    gn, kn, vn, inn, gsemn, msemn = bufs[(b + 1) % 3]
                    gl, kl, vl, il, gseml, mseml = bufs[(b + 2) % 3]

                    @pl.when(i < nbt)
                    def _():
                        gather_wait(iv, gv, gsem)

                        @pl.when(i + 1 < nbt)
                        def _():
                            meta_wait(i + 1, kn, vn, msemn)
                            gather_fire(kn, inn, gn, gsemn)

                        @pl.when(i + 2 < nbt)
                        def _():
                            meta_issue(i + 2, kl, vl, mseml)

                        process(gv, kv, vv)
                return carry
            lax.fori_loop(0, (nbt + 2) // 3, triple, 0)

            if has_prev:
                for q in range(4):
                    pltpu.sync_copy(
                        xprev_hbm.at[ch, pl.ds(base + q * (RPW // 4),
                                               RPW // 4)], p_v)

                    def recur(r, carry):
                        ar = q * (RPW // 4) + r
                        avs = [a_v[ar, pl.ds(j * 16, 16)] for j in range(8)]
                        pvs = [p_v[r, pl.ds(j * 16, 16)] for j in range(8)]
                        for j in range(8):
                            a_v[ar, pl.ds(j * 16, 16)] = 2.0 * avs[j] - pvs[j]
                        return carry
                    lax.fori_loop(0, RPW // 4, recur, 0)

            pltpu.sync_copy(a_v, y_hbm.at[ch, pl.ds(base, RPW)])
            return carry0

        lax.fori_loop(0, NCH, chunk, 0)

    return functools.partial(
        pl.kernel,
        out_type=jax.ShapeDtypeStruct((NCH, NPAD, CH), jnp.float32),
        mesh=mesh,
        scratch_types=[
            pltpu.VMEM((RPW, CH), jnp.float32),
            pltpu.VMEM((RPW // 4, CH), jnp.float32),
            pltpu.VMEM((EB, CH), jnp.float32),
            pltpu.VMEM((EB, CH), jnp.float32),
            pltpu.VMEM((EB, CH), jnp.float32),
            pltpu.VMEM((EB,), jnp.int32),
            pltpu.VMEM((EB,), jnp.int32),
            pltpu.VMEM((EB,), jnp.int32),
            pltpu.VMEM((EB,), jnp.float32),
            pltpu.VMEM((EB,), jnp.float32),
            pltpu.VMEM((EB,), jnp.float32),
            pltpu.VMEM((EB,), jnp.int32),
            pltpu.VMEM((EB,), jnp.int32),
            pltpu.VMEM((EB,), jnp.int32),
            pltpu.VMEM((48,), jnp.int32),
            pltpu.SemaphoreType.DMA,
            pltpu.SemaphoreType.DMA,
            pltpu.SemaphoreType.DMA,
            pltpu.SemaphoreType.DMA,
            pltpu.SemaphoreType.DMA,
            pltpu.SemaphoreType.DMA,
        ],
    )(body)


_spmm_first = _make_spmm(has_prev=False)
_spmm_next = _make_spmm(has_prev=True)


def _linear(xts, wts, b2):
    BN = 400
    nblk = N // BN

    def body(x0_ref, x1_ref, x2_ref, x3_ref, w_ref, b_ref, o_ref):
        acc = b_ref[...].astype(jnp.float32) + jnp.zeros((BN, COUT), jnp.float32)
        for t, xr in enumerate((x0_ref, x1_ref, x2_ref, x3_ref)):
            acc = acc + lax.dot_general(
                xr[0], w_ref[t],
                (((1,), (1,)), ((), ())),
                preferred_element_type=jnp.float32)
        o_ref[...] = acc.reshape(1, BN, COUT)

    return pl.pallas_call(
        body,
        grid=(NCH, nblk),
        in_specs=[pl.BlockSpec((1, BN, CH), lambda f, i: (f, i, 0))] * 4 + [
            pl.BlockSpec((K, COUT, CH), lambda f, i: (0, 0, 0)),
            pl.BlockSpec((1, COUT), lambda f, i: (0, 0)),
        ],
        out_specs=pl.BlockSpec((1, BN, COUT), lambda f, i: (f, i, 0)),
        out_shape=jax.ShapeDtypeStruct((NCH, N, COUT), jnp.float32),
    )(*xts, wts, b2)


def _gather_view(xcm):
    return xcm.reshape(NCH * NPAD, CH)


def kernel(x, L_values, L_indices, W, b):
    xcm0 = jnp.pad(x.transpose(1, 0, 2), ((0, 0), (0, NPAD - N), (0, 0)))

    rows = L_indices[0].astype(jnp.int32)
    cols = L_indices[1].astype(jnp.int32)
    packed = (rows << CBITS) | cols
    pk_s, vals_s = lax.sort((packed, L_values), dimension=0, num_keys=1)
    off = jnp.searchsorted(
        pk_s, jnp.arange(NW + 1, dtype=jnp.int32) * (RPW << CBITS)
    ).astype(jnp.int32)
    off = jnp.pad(off, (0, 48 - (NW + 1)), constant_values=E)

    pk_p = jnp.pad(pk_s, (0, EB), constant_values=NPAD << CBITS)
    vals_p = jnp.pad(vals_s, (0, EB))

    x1 = _spmm_first(_gather_view(xcm0), pk_p, vals_p, off)
    x2 = _spmm_next(_gather_view(x1), xcm0, pk_p, vals_p, off)
    x3 = _spmm_next(_gather_view(x2), x1, pk_p, vals_p, off)

    wts = W.reshape(COUT, CH, K).transpose(2, 0, 1)
    out = _linear([xcm0, x1, x2, x3], wts, b.reshape(1, COUT))
    return out.transpose(1, 0, 2)
